# Initial kernel scaffold; baseline (speedup 1.0000x reference)
import jax
import jax.numpy as jnp
from jax.experimental import pallas as pl


def kernel(x, table):
    raise NotImplementedError("write your pallas kernel here")


# speedup vs baseline: 9.1740x; 9.1740x over previous
import jax
import jax.numpy as jnp
from jax import lax
from jax.experimental import pallas as pl
from jax.experimental.pallas import tpu as pltpu
from jax.experimental.pallas import tpu_sc as plsc

IN_ROWS = 100001
D = 128
B = 4096
S = 200

NC = 2
NS = 16
NW = NC * NS
N = B * S
PER_W = N // NW
BLK = 128
NBLK = PER_W // BLK
NSLOT = 4


def _body(x_hbm, table_hbm, out_hbm, idx_v, g0, g1, g2, g3,
          sg0, sg1, sg2, sg3, so0, so1, so2, so3):
    wid = lax.axis_index("s") * NC + lax.axis_index("c")
    base = wid * PER_W

    gat = (g0, g1, g2, g3)
    semg = (sg0, sg1, sg2, sg3)
    semo = (so0, so1, so2, so3)

    pltpu.sync_copy(x_hbm.at[pl.ds(base, PER_W)], idx_v)

    def issue_gather(b, slot):
        pltpu.async_copy(
            table_hbm.at[idx_v.at[pl.ds(b * BLK, BLK)]], gat[slot],
            semg[slot])

    def wait_gather(slot):
        pltpu.make_async_copy(
            table_hbm.at[idx_v.at[pl.ds(0, BLK)]], gat[slot],
            semg[slot]).wait()

    def issue_out(b, slot):
        pltpu.async_copy(
            gat[slot], out_hbm.at[pl.ds(base + b * BLK, BLK)], semo[slot])

    def wait_out(slot):
        pltpu.make_async_copy(
            gat[slot], out_hbm.at[pl.ds(base, BLK)], semo[slot]).wait()

    issue_gather(0, 0)
    issue_gather(1, 1)

    def step_body(i, carry):
        for e in range(NSLOT):
            b = i * NSLOT + e
            nslot = (e + 2) % NSLOT
            wait_gather(e)
            issue_out(b, e)

            @pl.when(b >= 2)
            def _():
                wait_out(nslot)

            @pl.when(b < NBLK - 2)
            def _():
                issue_gather(b + 2, nslot)
        return carry

    lax.fori_loop(0, NBLK // NSLOT, step_body, 0, unroll=False)
    wait_out(2)
    wait_out(3)


def kernel(x, table):
    xf = x.astype(jnp.int32).reshape(N)
    run = pl.kernel(
        _body,
        out_type=jax.ShapeDtypeStruct((N, D), jnp.float32),
        mesh=plsc.VectorSubcoreMesh(core_axis_name="c", subcore_axis_name="s"),
        compiler_params=pltpu.CompilerParams(needs_layout_passes=False),
        scratch_types=(
            [pltpu.VMEM((PER_W,), jnp.int32)]
            + [pltpu.VMEM((BLK, D), jnp.float32) for _ in range(NSLOT)]
            + [pltpu.SemaphoreType.DMA for _ in range(2 * NSLOT)]
        ),
    )
    emb = run(xf, table)
    return jnp.swapaxes(emb.reshape(B, S, D), -1, -2)


# speedup vs baseline: 9.2012x; 1.0030x over previous
import jax
import jax.numpy as jnp
from jax import lax
from jax.experimental import pallas as pl
from jax.experimental.pallas import tpu as pltpu
from jax.experimental.pallas import tpu_sc as plsc

IN_ROWS = 100001
D = 128
B = 4096
S = 200

NC = 2
NS = 16
NW = NC * NS
N = B * S
PER_W = N // NW
BLK = 64
NBLK = PER_W // BLK
NSLOT = 8
LEAD = NSLOT // 2


def _body(x_hbm, table_hbm, out_hbm, idx_v, *rest):
    gat = rest[:NSLOT]
    semg = rest[NSLOT:2 * NSLOT]
    semo = rest[2 * NSLOT:]
    wid = lax.axis_index("s") * NC + lax.axis_index("c")
    base = wid * PER_W

    pltpu.sync_copy(x_hbm.at[pl.ds(base, PER_W)], idx_v)

    def issue_gather(b, slot):
        pltpu.async_copy(
            table_hbm.at[idx_v.at[pl.ds(b * BLK, BLK)]], gat[slot],
            semg[slot])

    def wait_gather(slot):
        pltpu.make_async_copy(
            table_hbm.at[idx_v.at[pl.ds(0, BLK)]], gat[slot],
            semg[slot]).wait()

    def issue_out(b, slot):
        pltpu.async_copy(
            gat[slot], out_hbm.at[pl.ds(base + b * BLK, BLK)], semo[slot])

    def wait_out(slot):
        pltpu.make_async_copy(
            gat[slot], out_hbm.at[pl.ds(base, BLK)], semo[slot]).wait()

    for k in range(LEAD):
        issue_gather(k, k)

    def step_body(i, carry):
        for e in range(NSLOT):
            b = i * NSLOT + e
            nslot = (e + LEAD) % NSLOT
            wait_gather(e)
            issue_out(b, e)

            @pl.when(b >= LEAD)
            def _():
                wait_out(nslot)

            @pl.when(b < NBLK - LEAD)
            def _():
                issue_gather(b + LEAD, nslot)
        return carry

    lax.fori_loop(0, NBLK // NSLOT, step_body, 0, unroll=False)
    for k in range(LEAD):
        wait_out((NBLK - LEAD + k) % NSLOT)


def kernel(x, table):
    xf = x.astype(jnp.int32).reshape(N)
    run = pl.kernel(
        _body,
        out_type=jax.ShapeDtypeStruct((N, D), jnp.float32),
        mesh=plsc.VectorSubcoreMesh(core_axis_name="c", subcore_axis_name="s"),
        compiler_params=pltpu.CompilerParams(needs_layout_passes=False),
        scratch_types=(
            [pltpu.VMEM((PER_W,), jnp.int32)]
            + [pltpu.VMEM((BLK, D), jnp.float32) for _ in range(NSLOT)]
            + [pltpu.SemaphoreType.DMA for _ in range(2 * NSLOT)]
        ),
    )
    emb = run(xf, table)
    return jnp.swapaxes(emb.reshape(B, S, D), -1, -2)
